# baseline (speedup 1.0000x reference)
import functools
import jax
import jax.numpy as jnp
from jax import lax
from jax.experimental import pallas as pl
from jax.experimental.pallas import tpu as pltpu
from jax.experimental.pallas import tpu_sc as plsc

_N_GROUP = 8
_TOPK_GROUP = 4
_TOP_K = 8
_SCALE = 2.5
_NEG = -1e30

_T = 8192
_E = 64
_TPW = 256
_CH = 16


def _mm_kernel(xa_ref, xb_ref, w_ref, b_ref, s_ref):
    hh = xa_ref.shape[1]
    logits_t = lax.dot_general(
        w_ref[:, :hh], xa_ref[...], (((1,), (1,)), ((), ())),
        preferred_element_type=jnp.float32)
    logits_t = logits_t + lax.dot_general(
        w_ref[:, hh:], xb_ref[...], (((1,), (1,)), ((), ())),
        preferred_element_type=jnp.float32)
    s_ref[...] = jax.nn.sigmoid(logits_t) + b_ref[...]


def _tree_max(vs):
    while len(vs) > 1:
        vs = [jnp.maximum(vs[i], vs[i + 1]) for i in range(0, len(vs) - 1, 2)] \
            + ([vs[-1]] if len(vs) % 2 else [])
    return vs[0]


def _sc_route(s_hbm, i_hbm, w_hbm, sv, iv, wv):
    wid = lax.axis_index("s") * 2 + lax.axis_index("c")
    base = wid * _TPW
    pltpu.sync_copy(s_hbm.at[:, pl.ds(base, _TPW)], sv)

    def chunk(c, carry):
        off = c * _CH
        v = [sv[e, pl.ds(off, _CH)] for e in range(_E)]

        gs = []
        for g in range(_N_GROUP):
            vg = v[8 * g:8 * g + 8]
            m1 = _tree_max(vg)
            eqs = [u == m1 for u in vg]
            cnt = jnp.zeros((_CH,), jnp.float32)
            for eq in eqs:
                cnt = cnt + jnp.where(eq, 1.0, 0.0)
            strict = _tree_max([jnp.where(eq, _NEG, u)
                                for eq, u in zip(eqs, vg)])
            gs.append(m1 + jnp.where(cnt >= 2.0, m1, strict))

        gsel = [jnp.zeros((_CH,), jnp.float32) for _ in range(_N_GROUP)]
        work = list(gs)
        for _ in range(_TOPK_GROUP):
            m = _tree_max(work)
            found = jnp.zeros((_CH,), jnp.float32)
            for g in range(_N_GROUP):
                eq0 = work[g] == m
                eqf = jnp.where(eq0, 1.0 - found, 0.0)
                take = eqf > 0.0
                gsel[g] = jnp.where(take, 1.0, gsel[g])
                found = jnp.where(eq0, 1.0, found)
                work[g] = jnp.where(take, _NEG, work[g])

        t = [jnp.where(gsel[e // 8] > 0.0, v[e], 0.0) for e in range(_E)]
        idxs, ws = [], []
        for _ in range(_TOP_K):
            m = _tree_max(t)
            found = jnp.zeros((_CH,), jnp.float32)
            fi = jnp.zeros((_CH,), jnp.float32)
            for e2 in range(_E):
                eq0 = t[e2] == m
                eqf = jnp.where(eq0, 1.0 - found, 0.0)
                take = eqf > 0.0
                fi = jnp.where(take, float(e2), fi)
                found = jnp.where(eq0, 1.0, found)
                t[e2] = jnp.where(take, _NEG, t[e2])
            idxs.append(fi)
            ws.append(m)

        dsum = ws[0]
        for u in ws[1:]:
            dsum = dsum + u
        scale = _SCALE / (dsum + 1e-20)
        for k in range(_TOP_K):
            iv[k, pl.ds(off, _CH)] = idxs[k].astype(jnp.int32)
            wv[k, pl.ds(off, _CH)] = ws[k] * scale
        return carry

    lax.fori_loop(0, _TPW // _CH, chunk, 0)
    pltpu.sync_copy(iv, i_hbm.at[:, pl.ds(base, _TPW)])
    pltpu.sync_copy(wv, w_hbm.at[:, pl.ds(base, _TPW)])


def kernel(hidden_states, weight, e_score_correction_bias):
    bsz, seq, h = hidden_states.shape
    n_experts = weight.shape[0]
    t = bsz * seq
    bt = 1024
    hh = h // 2

    x2 = hidden_states.reshape(t, h)
    w = weight.astype(jnp.float32)
    b2 = e_score_correction_bias.reshape(n_experts, 1).astype(jnp.float32)

    scores_t = pl.pallas_call(
        _mm_kernel,
        grid=(t // bt,),
        in_specs=[
            pl.BlockSpec((bt, hh), lambda i: (i, 0)),
            pl.BlockSpec((bt, hh), lambda i: (i, 1)),
            pl.BlockSpec((n_experts, h), lambda i: (0, 0)),
            pl.BlockSpec((n_experts, 1), lambda i: (0, 0)),
        ],
        out_specs=pl.BlockSpec((n_experts, bt), lambda i: (0, i)),
        out_shape=jax.ShapeDtypeStruct((n_experts, t), jnp.float32),
        compiler_params=pltpu.CompilerParams(
            dimension_semantics=("arbitrary",),
        ),
    )(x2, x2, w, b2)

    mesh = plsc.VectorSubcoreMesh(
        core_axis_name="c", subcore_axis_name="s")
    idx8, w8 = pl.kernel(
        _sc_route,
        out_type=[
            jax.ShapeDtypeStruct((_TOP_K, t), jnp.int32),
            jax.ShapeDtypeStruct((_TOP_K, t), jnp.float32),
        ],
        mesh=mesh,
        scratch_types=[
            pltpu.VMEM((n_experts, _TPW), jnp.float32),
            pltpu.VMEM((_TOP_K, _TPW), jnp.int32),
            pltpu.VMEM((_TOP_K, _TPW), jnp.float32),
        ],
    )(scores_t)

    return idx8.T, w8.T


# speedup vs baseline: 2.2756x; 2.2756x over previous
import jax
import jax.numpy as jnp
from jax.experimental import pallas as pl
from jax.experimental.pallas import tpu as pltpu

_N_GROUP = 8
_TOPK_GROUP = 4
_TOP_K = 8
_SCALE = 2.5
_NEG = -1e30


def _routing(sfc, scores_t):
    e, bt = sfc.shape
    spg = e // _N_GROUP

    g3 = sfc.reshape(_N_GROUP, spg, bt)
    m1 = jnp.max(g3, axis=1, keepdims=True)
    m1b = jnp.broadcast_to(m1, g3.shape)
    eq = g3 == m1b
    cnt = jnp.sum(eq.astype(jnp.float32), axis=1, keepdims=True)
    strict = jnp.max(jnp.where(eq, _NEG, g3), axis=1, keepdims=True)
    m2 = jnp.where(cnt >= 2.0, m1, strict)
    gs = m1 + m2

    growf = jax.lax.broadcasted_iota(
        jnp.int32, (_N_GROUP, 1, bt), 0).astype(jnp.float32)
    gidf = (jax.lax.broadcasted_iota(
        jnp.int32, (e, bt), 0) // spg).astype(jnp.float32)
    t8 = gs
    gmask = jnp.zeros((e, bt), dtype=jnp.bool_)
    for _ in range(_TOPK_GROUP):
        m = jnp.max(t8, axis=0, keepdims=True)
        fi = jnp.min(jnp.where(t8 == m, growf, float(_N_GROUP)),
                     axis=0, keepdims=True)
        fi2 = fi.reshape(1, bt)
        gmask = gmask | (gidf == fi2)
        t8 = jnp.where(growf == fi, _NEG, t8)

    tmp = jnp.where(gmask, sfc, 0.0)

    frow = jax.lax.broadcasted_iota(
        jnp.int32, (e, bt), 0).astype(jnp.float32)
    row8 = jax.lax.broadcasted_iota(
        jnp.int32, (_TOP_K, bt), 0).astype(jnp.float32)
    acc_i = jnp.zeros((_TOP_K, bt), dtype=jnp.float32)
    acc_w = jnp.zeros((_TOP_K, bt), dtype=jnp.float32)
    t = tmp
    for k in range(_TOP_K):
        m = jnp.max(t, axis=0, keepdims=True)
        fi = jnp.min(jnp.where(t == m, frow, float(e)),
                     axis=0, keepdims=True)
        acc_i = jnp.where(row8 == float(k), fi, acc_i)
        acc_w = jnp.where(row8 == float(k), m, acc_w)
        t = jnp.where(frow == fi, _NEG, t)

    denom = jnp.sum(acc_w, axis=0, keepdims=True) + 1e-20
    w_out = acc_w * (_SCALE / denom)
    return acc_i.astype(jnp.int32).T, w_out.T


def _gate_kernel(x_ref, w_ref, b_ref, idx_ref, w_out_ref):
    logits_t = jax.lax.dot_general(
        w_ref[...], x_ref[...], (((1,), (1,)), ((), ())),
        preferred_element_type=jnp.float32)
    scores_t = jax.nn.sigmoid(logits_t)
    sfc = scores_t + b_ref[...]
    idx, wts = _routing(sfc, scores_t)
    idx_ref[...] = idx
    w_out_ref[...] = wts


def kernel(hidden_states, weight, e_score_correction_bias):
    bsz, seq, h = hidden_states.shape
    n_experts = weight.shape[0]
    t = bsz * seq
    bt = 1024

    x2 = hidden_states.reshape(t, h)
    w = weight.astype(jnp.float32)
    b2 = e_score_correction_bias.reshape(n_experts, 1).astype(jnp.float32)

    idx, wts = pl.pallas_call(
        _gate_kernel,
        grid=(t // bt,),
        in_specs=[
            pl.BlockSpec((bt, h), lambda i: (i, 0)),
            pl.BlockSpec((n_experts, h), lambda i: (0, 0)),
            pl.BlockSpec((n_experts, 1), lambda i: (0, 0)),
        ],
        out_specs=[
            pl.BlockSpec((bt, _TOP_K), lambda i: (i, 0)),
            pl.BlockSpec((bt, _TOP_K), lambda i: (i, 0)),
        ],
        out_shape=[
            jax.ShapeDtypeStruct((t, _TOP_K), jnp.int32),
            jax.ShapeDtypeStruct((t, _TOP_K), jnp.float32),
        ],
        compiler_params=pltpu.CompilerParams(
            dimension_semantics=("arbitrary",),
        ),
    )(x2, w, b2)
    return idx, wts
